# baseline (speedup 1.0000x reference)
import jax
import jax.numpy as jnp
from jax.experimental import pallas as pl
from jax.experimental.pallas import tpu as pltpu

_BN = 1024


def _split2_bf16(v):
    hi = v.astype(jnp.bfloat16)
    mid = (v - hi.astype(jnp.float32)).astype(jnp.bfloat16)
    return hi, mid


def _chamfer_body(x1_ref, x2t_ref, d1_ref, d2_ref):
    i = pl.program_id(1)

    x1 = x1_ref[0]
    x2t = x2t_ref[0]

    n1 = jnp.sum(x1 * x1, axis=1, keepdims=True)
    n2 = jnp.sum(x2t * x2t, axis=0, keepdims=True)

    n1h, n1m = _split2_bf16(n1)
    n2h, n2m = _split2_bf16(n2)
    bn = x1.shape[0]
    ones_col = jnp.ones((bn, 2), jnp.bfloat16)
    ones_row = jnp.ones((2, x2t.shape[1]), jnp.bfloat16)

    lhs = jnp.concatenate(
        [(-2.0 * x1).astype(jnp.bfloat16), ones_col, n1h, n1m], axis=1)
    rhs = jnp.concatenate(
        [x2t.astype(jnp.bfloat16), n2h, n2m, ones_row], axis=0)

    d2 = jnp.dot(lhs, rhs, preferred_element_type=jnp.float32)

    m = x2t.shape[1]
    part = d2[:, 0:128]
    for k in range(1, m // 128):
        part = jnp.minimum(part, d2[:, k * 128:(k + 1) * 128])
    d1_ref[0, 0, :] = jnp.maximum(jnp.min(part.T, axis=0), 0.0)

    col_min = jnp.maximum(jnp.min(d2, axis=0, keepdims=True), 0.0)[None]

    @pl.when(i == 0)
    def _():
        d2_ref[...] = col_min

    @pl.when(i > 0)
    def _():
        d2_ref[...] = jnp.minimum(d2_ref[...], col_min)


def kernel(xyz1, xyz2):
    xyz1 = xyz1.astype(jnp.float32)
    xyz2 = xyz2.astype(jnp.float32)
    B, N, _ = xyz1.shape
    _, M, _ = xyz2.shape
    x2t = jnp.swapaxes(xyz2, 1, 2)

    grid = (B, N // _BN)
    dist1, dist2 = pl.pallas_call(
        _chamfer_body,
        grid=grid,
        in_specs=[
            pl.BlockSpec((1, _BN, 3), lambda b, i: (b, i, 0)),
            pl.BlockSpec((1, 3, M), lambda b, i: (b, 0, 0)),
        ],
        out_specs=[
            pl.BlockSpec((1, 1, _BN), lambda b, i: (b, 0, i)),
            pl.BlockSpec((1, 1, M), lambda b, i: (b, 0, 0)),
        ],
        out_shape=[
            jax.ShapeDtypeStruct((B, 1, N), jnp.float32),
            jax.ShapeDtypeStruct((B, 1, M), jnp.float32),
        ],
        compiler_params=pltpu.CompilerParams(
            dimension_semantics=("parallel", "arbitrary"),
        ),
    )(xyz1, x2t)
    return (dist1[:, 0, :], dist2[:, 0, :])


# speedup vs baseline: 1.0012x; 1.0012x over previous
import jax
import jax.numpy as jnp
from jax.experimental import pallas as pl
from jax.experimental.pallas import tpu as pltpu

_BN = 1024


def _split3_bf16(v):
    hi = v.astype(jnp.bfloat16)
    r = v - hi.astype(jnp.float32)
    mid = r.astype(jnp.bfloat16)
    lo = (r - mid.astype(jnp.float32)).astype(jnp.bfloat16)
    return hi, mid, lo


def _chamfer_body(x1_ref, x2t_ref, d1_ref, d2_ref):
    i = pl.program_id(1)

    x1 = x1_ref[0]
    x2t = x2t_ref[0]

    n1 = jnp.sum(x1 * x1, axis=1, keepdims=True)
    n2 = jnp.sum(x2t * x2t, axis=0, keepdims=True)

    n1h, n1m, n1l = _split3_bf16(n1)
    n2h, n2m, n2l = _split3_bf16(n2)
    bn = x1.shape[0]
    ones_col = jnp.ones((bn, 3), jnp.bfloat16)
    ones_row = jnp.ones((3, x2t.shape[1]), jnp.bfloat16)

    lhs = jnp.concatenate(
        [(-2.0 * x1).astype(jnp.bfloat16), ones_col, n1h, n1m, n1l], axis=1)
    rhs = jnp.concatenate(
        [x2t.astype(jnp.bfloat16), n2h, n2m, n2l, ones_row], axis=0)

    d2 = jnp.dot(lhs, rhs, preferred_element_type=jnp.float32)

    m = x2t.shape[1]
    part = d2[:, 0:128]
    for k in range(1, m // 128):
        part = jnp.minimum(part, d2[:, k * 128:(k + 1) * 128])
    d1_ref[0, 0, :] = jnp.maximum(jnp.min(part.T, axis=0), 0.0)

    col_min = jnp.maximum(jnp.min(d2, axis=0, keepdims=True), 0.0)[None]

    @pl.when(i == 0)
    def _():
        d2_ref[...] = col_min

    @pl.when(i > 0)
    def _():
        d2_ref[...] = jnp.minimum(d2_ref[...], col_min)


def kernel(xyz1, xyz2):
    xyz1 = xyz1.astype(jnp.float32)
    xyz2 = xyz2.astype(jnp.float32)
    B, N, _ = xyz1.shape
    _, M, _ = xyz2.shape
    x2t = jnp.swapaxes(xyz2, 1, 2)

    grid = (B, N // _BN)
    dist1, dist2 = pl.pallas_call(
        _chamfer_body,
        grid=grid,
        in_specs=[
            pl.BlockSpec((1, _BN, 3), lambda b, i: (b, i, 0)),
            pl.BlockSpec((1, 3, M), lambda b, i: (b, 0, 0)),
        ],
        out_specs=[
            pl.BlockSpec((1, 1, _BN), lambda b, i: (b, 0, i)),
            pl.BlockSpec((1, 1, M), lambda b, i: (b, 0, 0)),
        ],
        out_shape=[
            jax.ShapeDtypeStruct((B, 1, N), jnp.float32),
            jax.ShapeDtypeStruct((B, 1, M), jnp.float32),
        ],
        compiler_params=pltpu.CompilerParams(
            dimension_semantics=("parallel", "arbitrary"),
        ),
    )(xyz1, x2t)
    return (dist1[:, 0, :], dist2[:, 0, :])
